# Initial kernel scaffold; baseline (speedup 1.0000x reference)
import jax
import jax.numpy as jnp
from jax.experimental import pallas as pl


def kernel(x, W0, W1, W2, W3, W4, W5, W6, W7, W8, W9, W10, W11, W12, W13, W14, W15, W16, W17, W18, W19, W20, W21, W22, W23, W24, W25):
    raise NotImplementedError("write your pallas kernel here")


# speedup vs baseline: 4.0920x; 4.0920x over previous
import functools

import jax
import jax.numpy as jnp
from jax import lax
from jax.experimental import pallas as pl
from jax.experimental.pallas import tpu as pltpu
from jax.experimental.pallas import tpu_sc as plsc

_NUM_FEATURES = 26
_VOCAB = 128
_BATCH = 16384
_MAX_NORM = 1.0

_ROWS = _BATCH * _NUM_FEATURES
_TABLE_ROWS = _NUM_FEATURES * _VOCAB

_NC, _NS, _L = 2, 16, 16
_NW = _NC * _NS
_PER_W = _ROWS // _NW
_CHUNK = 128
_NBUF = 4
_CHUNKS_PER_W = _PER_W // _CHUNK
_OUTER = _CHUNKS_PER_W // _NBUF


def _scale_body(t_ref, o_ref):
    rows = t_ref[...]
    norm = jnp.sqrt(jnp.sum(rows * rows, axis=1, keepdims=True))
    scale = jnp.minimum(1.0, _MAX_NORM / jnp.maximum(norm, 1e-7))
    o_ref[...] = rows * scale


_scale_call = pl.pallas_call(
    _scale_body,
    out_shape=jax.ShapeDtypeStruct((_TABLE_ROWS, _VOCAB), jnp.float32),
)


def _sc_gather_body(x_hbm, table_hbm, out_hbm, xv, gidx, b0, b1, b2, b3,
                    s0, s1, s2, s3):
    bufs = (b0, b1, b2, b3)
    sems = (s0, s1, s2, s3)
    wid = lax.axis_index("s") * _NC + lax.axis_index("c")
    base = wid * _PER_W

    pltpu.sync_copy(x_hbm.at[pl.ds(base, _PER_W)], xv)

    lanes = lax.iota(jnp.int32, _L)

    def idx_body(j, carry):
        r = j * _L
        off = lax.rem(base + r + lanes, _NUM_FEATURES) * _VOCAB
        gidx[pl.ds(r, _L)] = xv[pl.ds(r, _L)] + off
        return carry

    lax.fori_loop(0, _PER_W // _L, idx_body, 0, unroll=4)

    def chunk_body(p, carry):
        copies = []
        for b in range(_NBUF):
            c = p * _NBUF + b
            copies.append(
                pltpu.async_copy(
                    table_hbm.at[gidx.at[pl.ds(c * _CHUNK, _CHUNK)]],
                    bufs[b],
                    sems[b],
                )
            )
        for b in range(_NBUF):
            c = p * _NBUF + b
            copies[b].wait()
            pltpu.sync_copy(bufs[b], out_hbm.at[pl.ds(base + c * _CHUNK, _CHUNK)])
        return carry

    lax.fori_loop(0, _OUTER, chunk_body, 0)


@functools.cache
def _make_sc_gather():
    mesh = plsc.VectorSubcoreMesh(core_axis_name="c", subcore_axis_name="s")
    return pl.kernel(
        _sc_gather_body,
        mesh=mesh,
        out_type=jax.ShapeDtypeStruct((_ROWS, _VOCAB), jnp.float32),
        scratch_types=[
            pltpu.VMEM((_PER_W,), jnp.int32),
            pltpu.VMEM((_PER_W,), jnp.int32),
            pltpu.VMEM((_CHUNK, _VOCAB), jnp.float32),
            pltpu.VMEM((_CHUNK, _VOCAB), jnp.float32),
            pltpu.VMEM((_CHUNK, _VOCAB), jnp.float32),
            pltpu.VMEM((_CHUNK, _VOCAB), jnp.float32),
            pltpu.SemaphoreType.DMA,
            pltpu.SemaphoreType.DMA,
            pltpu.SemaphoreType.DMA,
            pltpu.SemaphoreType.DMA,
        ],
    )


def kernel(x, W0, W1, W2, W3, W4, W5, W6, W7, W8, W9, W10, W11, W12, W13,
           W14, W15, W16, W17, W18, W19, W20, W21, W22, W23, W24, W25):
    Ws = [W0, W1, W2, W3, W4, W5, W6, W7, W8, W9, W10, W11, W12, W13,
          W14, W15, W16, W17, W18, W19, W20, W21, W22, W23, W24, W25]
    table = jnp.concatenate(Ws, axis=0)
    scaled = _scale_call(table)
    x_flat = x.astype(jnp.int32).reshape(-1)
    rows = _make_sc_gather()(x_flat, scaled)
    return rows.reshape(_BATCH, _NUM_FEATURES * _VOCAB)


# speedup vs baseline: 4.0923x; 1.0001x over previous
import functools

import jax
import jax.numpy as jnp
from jax import lax
from jax.experimental import pallas as pl
from jax.experimental.pallas import tpu as pltpu
from jax.experimental.pallas import tpu_sc as plsc

_NUM_FEATURES = 26
_VOCAB = 128
_BATCH = 16384
_MAX_NORM = 1.0

_ROWS = _BATCH * _NUM_FEATURES
_TABLE_ROWS = _NUM_FEATURES * _VOCAB

_NC, _NS, _L = 2, 16, 16
_NW = _NC * _NS
_PER_W = _ROWS // _NW
_CHUNK = 128
_NBUF = 4
_CHUNKS_PER_W = _PER_W // _CHUNK
_OUTER = _CHUNKS_PER_W // _NBUF


def _scale_body(t_ref, o_ref):
    rows = t_ref[...]
    norm = jnp.sqrt(jnp.sum(rows * rows, axis=1, keepdims=True))
    scale = jnp.minimum(1.0, _MAX_NORM / jnp.maximum(norm, 1e-7))
    o_ref[...] = rows * scale


_scale_call = pl.pallas_call(
    _scale_body,
    out_shape=jax.ShapeDtypeStruct((_TABLE_ROWS, _VOCAB), jnp.float32),
)


def _sc_gather_body(x_hbm, table_hbm, out_hbm, xv, gidx, b0, b1, b2, b3,
                    s0, s1, s2, s3, t0, t1, t2, t3):
    bufs = (b0, b1, b2, b3)
    sems = (s0, s1, s2, s3)
    osems = (t0, t1, t2, t3)
    wid = lax.axis_index("s") * _NC + lax.axis_index("c")
    base = wid * _PER_W

    pltpu.sync_copy(x_hbm.at[pl.ds(base, _PER_W)], xv)

    lanes = lax.iota(jnp.int32, _L)

    def idx_body(j, carry):
        r = j * _L
        off = lax.rem(base + r + lanes, _NUM_FEATURES) * _VOCAB
        gidx[pl.ds(r, _L)] = xv[pl.ds(r, _L)] + off
        return carry

    lax.fori_loop(0, _PER_W // _L, idx_body, 0, unroll=4)

    def _drain_out(b):
        pltpu.make_async_copy(
            bufs[b], out_hbm.at[pl.ds(base, _CHUNK)], osems[b]).wait()

    def chunk_body(p, carry):
        @pl.when(p > 0)
        def _():
            for b in range(_NBUF):
                _drain_out(b)

        copies = []
        for b in range(_NBUF):
            c = p * _NBUF + b
            copies.append(
                pltpu.async_copy(
                    table_hbm.at[gidx.at[pl.ds(c * _CHUNK, _CHUNK)]],
                    bufs[b],
                    sems[b],
                )
            )
        for b in range(_NBUF):
            c = p * _NBUF + b
            copies[b].wait()
            pltpu.async_copy(
                bufs[b], out_hbm.at[pl.ds(base + c * _CHUNK, _CHUNK)], osems[b])
        return carry

    lax.fori_loop(0, _OUTER, chunk_body, 0)
    for b in range(_NBUF):
        _drain_out(b)


@functools.cache
def _make_sc_gather():
    mesh = plsc.VectorSubcoreMesh(core_axis_name="c", subcore_axis_name="s")
    return pl.kernel(
        _sc_gather_body,
        mesh=mesh,
        out_type=jax.ShapeDtypeStruct((_ROWS, _VOCAB), jnp.float32),
        scratch_types=[
            pltpu.VMEM((_PER_W,), jnp.int32),
            pltpu.VMEM((_PER_W,), jnp.int32),
            pltpu.VMEM((_CHUNK, _VOCAB), jnp.float32),
            pltpu.VMEM((_CHUNK, _VOCAB), jnp.float32),
            pltpu.VMEM((_CHUNK, _VOCAB), jnp.float32),
            pltpu.VMEM((_CHUNK, _VOCAB), jnp.float32),
            pltpu.SemaphoreType.DMA,
            pltpu.SemaphoreType.DMA,
            pltpu.SemaphoreType.DMA,
            pltpu.SemaphoreType.DMA,
            pltpu.SemaphoreType.DMA,
            pltpu.SemaphoreType.DMA,
            pltpu.SemaphoreType.DMA,
            pltpu.SemaphoreType.DMA,
        ],
    )


def kernel(x, W0, W1, W2, W3, W4, W5, W6, W7, W8, W9, W10, W11, W12, W13,
           W14, W15, W16, W17, W18, W19, W20, W21, W22, W23, W24, W25):
    Ws = [W0, W1, W2, W3, W4, W5, W6, W7, W8, W9, W10, W11, W12, W13,
          W14, W15, W16, W17, W18, W19, W20, W21, W22, W23, W24, W25]
    table = jnp.concatenate(Ws, axis=0)
    scaled = _scale_call(table)
    x_flat = x.astype(jnp.int32).reshape(-1)
    rows = _make_sc_gather()(x_flat, scaled)
    return rows.reshape(_BATCH, _NUM_FEATURES * _VOCAB)


# speedup vs baseline: 8.5533x; 2.0901x over previous
import functools

import jax
import jax.numpy as jnp
from jax import lax
from jax.experimental import pallas as pl
from jax.experimental.pallas import tpu as pltpu
from jax.experimental.pallas import tpu_sc as plsc

_NUM_FEATURES = 26
_VOCAB = 128
_BATCH = 16384
_MAX_NORM = 1.0

_ROWS = _BATCH * _NUM_FEATURES
_TABLE_ROWS = _NUM_FEATURES * _VOCAB
_WIDTH = _NUM_FEATURES * _VOCAB

_NC, _NS, _L = 2, 16, 16
_NW = _NC * _NS
_PER_W = _ROWS // _NW
_BATCH_PER_W = _BATCH // _NW
_CHUNK = 104
_BROWS = _CHUNK // _NUM_FEATURES
_NBUF = 4
_CHUNKS_PER_W = _PER_W // _CHUNK
_OUTER = _CHUNKS_PER_W // _NBUF


def _scale_body(*refs):
    o_ref = refs[-1]
    for i in range(_NUM_FEATURES):
        rows = refs[i][...]
        norm = jnp.sqrt(jnp.sum(rows * rows, axis=1, keepdims=True))
        scale = jnp.minimum(1.0, _MAX_NORM / jnp.maximum(norm, 1e-7))
        o_ref[pl.ds(i * _VOCAB, _VOCAB), :] = rows * scale


_scale_call = pl.pallas_call(
    _scale_body,
    out_shape=jax.ShapeDtypeStruct((_TABLE_ROWS, _VOCAB), jnp.float32),
)


def _sc_gather_body(x_hbm, table_hbm, out_hbm, xv, gidx, b0, b1, b2, b3,
                    s0, s1, s2, s3, t0, t1, t2, t3):
    bufs = (b0, b1, b2, b3)
    sems = (s0, s1, s2, s3)
    osems = (t0, t1, t2, t3)
    wid = lax.axis_index("s") * _NC + lax.axis_index("c")
    base = wid * _PER_W
    brow = wid * _BATCH_PER_W

    pltpu.sync_copy(x_hbm.at[pl.ds(base, _PER_W)], xv)

    lanes = lax.iota(jnp.int32, _L)

    def idx_body(j, carry):
        r = j * _L
        off = lax.rem(base + r + lanes, _NUM_FEATURES) * _VOCAB
        gidx[pl.ds(r, _L)] = xv[pl.ds(r, _L)] + off
        return carry

    lax.fori_loop(0, _PER_W // _L, idx_body, 0, unroll=4)

    def _drain_out(b):
        pltpu.make_async_copy(
            bufs[b].reshape(_BROWS, _WIDTH),
            out_hbm.at[pl.ds(brow, _BROWS)], osems[b]).wait()

    def chunk_body(p, carry):
        @pl.when(p > 0)
        def _():
            for b in range(_NBUF):
                _drain_out(b)

        copies = []
        for b in range(_NBUF):
            c = p * _NBUF + b
            copies.append(
                pltpu.async_copy(
                    table_hbm.at[gidx.at[pl.ds(c * _CHUNK, _CHUNK)]],
                    bufs[b],
                    sems[b],
                )
            )
        for b in range(_NBUF):
            c = p * _NBUF + b
            copies[b].wait()
            pltpu.async_copy(
                bufs[b].reshape(_BROWS, _WIDTH),
                out_hbm.at[pl.ds(brow + c * _BROWS, _BROWS)],
                osems[b])
        return carry

    lax.fori_loop(0, _OUTER, chunk_body, 0)
    for b in range(_NBUF):
        _drain_out(b)


@functools.cache
def _make_sc_gather():
    mesh = plsc.VectorSubcoreMesh(core_axis_name="c", subcore_axis_name="s")
    return pl.kernel(
        _sc_gather_body,
        mesh=mesh,
        out_type=jax.ShapeDtypeStruct((_BATCH, _WIDTH), jnp.float32),
        scratch_types=[
            pltpu.VMEM((_PER_W,), jnp.int32),
            pltpu.VMEM((_PER_W,), jnp.int32),
            pltpu.VMEM((_CHUNK, _VOCAB), jnp.float32),
            pltpu.VMEM((_CHUNK, _VOCAB), jnp.float32),
            pltpu.VMEM((_CHUNK, _VOCAB), jnp.float32),
            pltpu.VMEM((_CHUNK, _VOCAB), jnp.float32),
            pltpu.SemaphoreType.DMA,
            pltpu.SemaphoreType.DMA,
            pltpu.SemaphoreType.DMA,
            pltpu.SemaphoreType.DMA,
            pltpu.SemaphoreType.DMA,
            pltpu.SemaphoreType.DMA,
            pltpu.SemaphoreType.DMA,
            pltpu.SemaphoreType.DMA,
        ],
    )


def kernel(x, W0, W1, W2, W3, W4, W5, W6, W7, W8, W9, W10, W11, W12, W13,
           W14, W15, W16, W17, W18, W19, W20, W21, W22, W23, W24, W25):
    Ws = [W0, W1, W2, W3, W4, W5, W6, W7, W8, W9, W10, W11, W12, W13,
          W14, W15, W16, W17, W18, W19, W20, W21, W22, W23, W24, W25]
    scaled = _scale_call(*Ws)
    x_flat = x.astype(jnp.int32).reshape(-1)
    return _make_sc_gather()(x_flat, scaled)


# speedup vs baseline: 8.7576x; 1.0239x over previous
import functools

import jax
import jax.numpy as jnp
from jax import lax
from jax.experimental import pallas as pl
from jax.experimental.pallas import tpu as pltpu
from jax.experimental.pallas import tpu_sc as plsc

_NUM_FEATURES = 26
_VOCAB = 128
_BATCH = 16384
_MAX_NORM = 1.0

_ROWS = _BATCH * _NUM_FEATURES
_TABLE_ROWS = _NUM_FEATURES * _VOCAB
_WIDTH = _NUM_FEATURES * _VOCAB

_NC, _NS, _L = 2, 16, 16
_NW = _NC * _NS
_PER_W = _ROWS // _NW
_BATCH_PER_W = _BATCH // _NW
_CHUNK = 104
_BROWS = _CHUNK // _NUM_FEATURES
_NBUF = 8
_CHUNKS_PER_W = _PER_W // _CHUNK
_OUTER = _CHUNKS_PER_W // _NBUF


def _scale_body(*refs):
    o_ref = refs[-1]
    for i in range(_NUM_FEATURES):
        rows = refs[i][...]
        norm = jnp.sqrt(jnp.sum(rows * rows, axis=1, keepdims=True))
        scale = jnp.minimum(1.0, _MAX_NORM / jnp.maximum(norm, 1e-7))
        o_ref[pl.ds(i * _VOCAB, _VOCAB), :] = rows * scale


_scale_call = pl.pallas_call(
    _scale_body,
    out_shape=jax.ShapeDtypeStruct((_TABLE_ROWS, _VOCAB), jnp.float32),
)


def _sc_gather_body(x_hbm, table_hbm, out_hbm, idxv,
                    b0, b1, b2, b3, b4, b5, b6, b7,
                    s0, s1, s2, s3, s4, s5, s6, s7,
                    t0, t1, t2, t3, t4, t5, t6, t7):
    bufs = (b0, b1, b2, b3, b4, b5, b6, b7)
    sems = (s0, s1, s2, s3, s4, s5, s6, s7)
    osems = (t0, t1, t2, t3, t4, t5, t6, t7)
    wid = lax.axis_index("s") * _NC + lax.axis_index("c")
    base = wid * _PER_W
    brow = wid * _BATCH_PER_W

    pltpu.sync_copy(x_hbm.at[pl.ds(base, _PER_W)], idxv)

    lanes = lax.iota(jnp.int32, _L)

    def idx_body(j, carry):
        r = j * _L
        off = lax.rem(base + r + lanes, _NUM_FEATURES) * _VOCAB
        idxv[pl.ds(r, _L)] = idxv[pl.ds(r, _L)] + off
        return carry

    lax.fori_loop(0, _PER_W // _L, idx_body, 0, unroll=4)

    def _drain_out(b):
        pltpu.make_async_copy(
            bufs[b].reshape(_BROWS, _WIDTH),
            out_hbm.at[pl.ds(brow, _BROWS)], osems[b]).wait()

    def chunk_body(p, carry):
        copies = []
        for b in range(_NBUF):
            @pl.when(p > 0)
            def _():
                _drain_out(b)

            c = p * _NBUF + b
            copies.append(
                pltpu.async_copy(
                    table_hbm.at[idxv.at[pl.ds(c * _CHUNK, _CHUNK)]],
                    bufs[b],
                    sems[b],
                )
            )
        for b in range(_NBUF):
            c = p * _NBUF + b
            copies[b].wait()
            pltpu.async_copy(
                bufs[b].reshape(_BROWS, _WIDTH),
                out_hbm.at[pl.ds(brow + c * _BROWS, _BROWS)],
                osems[b])
        return carry

    lax.fori_loop(0, _OUTER, chunk_body, 0)
    for b in range(_NBUF):
        _drain_out(b)


@functools.cache
def _make_sc_gather():
    mesh = plsc.VectorSubcoreMesh(core_axis_name="c", subcore_axis_name="s")
    return pl.kernel(
        _sc_gather_body,
        mesh=mesh,
        out_type=jax.ShapeDtypeStruct((_BATCH, _WIDTH), jnp.float32),
        scratch_types=[
            pltpu.VMEM((_PER_W,), jnp.int32),
        ] + [pltpu.VMEM((_CHUNK, _VOCAB), jnp.float32)] * _NBUF
          + [pltpu.SemaphoreType.DMA] * (2 * _NBUF),
    )


def kernel(x, W0, W1, W2, W3, W4, W5, W6, W7, W8, W9, W10, W11, W12, W13,
           W14, W15, W16, W17, W18, W19, W20, W21, W22, W23, W24, W25):
    Ws = [W0, W1, W2, W3, W4, W5, W6, W7, W8, W9, W10, W11, W12, W13,
          W14, W15, W16, W17, W18, W19, W20, W21, W22, W23, W24, W25]
    scaled = _scale_call(*Ws)
    x_flat = x.astype(jnp.int32).reshape(-1)
    return _make_sc_gather()(x_flat, scaled)
